# baseline (speedup 1.0000x reference)
import jax
import jax.numpy as jnp
from jax.experimental import pallas as pl
from jax.experimental.pallas import tpu as pltpu

_G = 52
_GG = _G * _G
_C = 25
_STRIDE = 8.0


def _body(anchor_ref, x_ref, out_ref):
    xa = x_ref[0, 0].reshape(_C, _GG)

    cidx = jax.lax.broadcasted_iota(jnp.int32, (_C, _GG), 0)
    p = jax.lax.broadcasted_iota(
        jnp.int32, (_C, _GG), 1).astype(jnp.float32)
    gy = jnp.floor((p + 0.5) * (1.0 / _G))
    gx = p - _G * gy

    a = pl.program_id(1)
    aw = anchor_ref[a, 0]
    ah = anchor_ref[a, 1]

    sig = jax.nn.sigmoid(xa)
    ex = jnp.exp(xa)

    is_w = cidx == 2
    is_h = cidx == 3
    base = jnp.where(is_w | is_h, ex, sig)
    add = jnp.where(cidx == 0, gx, jnp.where(cidx == 1, gy, 0.0))
    scale = jnp.where(cidx < 2, _STRIDE,
                      jnp.where(is_w, aw, jnp.where(is_h, ah, 1.0)))
    y = (base + add) * scale
    out_ref[0] = y.T


def kernel(x, anchor_wh):
    B = x.shape[0]
    x5 = x.reshape(B, 3, _C, _G, _G)
    out = pl.pallas_call(
        _body,
        grid=(B, 3),
        in_specs=[
            pl.BlockSpec(memory_space=pltpu.SMEM),
            pl.BlockSpec((1, 1, _C, _G, _G), lambda b, a: (b, a, 0, 0, 0)),
        ],
        out_specs=pl.BlockSpec((1, _GG, _C), lambda b, a: (b, a, 0)),
        out_shape=jax.ShapeDtypeStruct((B, 3 * _GG, _C), jnp.float32),
        compiler_params=pltpu.CompilerParams(
            dimension_semantics=("parallel", "arbitrary"),
        ),
    )(anchor_wh, x5)
    return out


# speedup vs baseline: 1.0237x; 1.0237x over previous
import jax
import jax.numpy as jnp
from jax.experimental import pallas as pl
from jax.experimental.pallas import tpu as pltpu

_G = 52
_GG = _G * _G
_C = 25
_STRIDE = 8.0


def _body(anchor_ref, x_ref, out_ref):
    xa = x_ref[0].reshape(_C, _GG)

    cidx = jax.lax.broadcasted_iota(jnp.int32, (_C, _GG), 0)
    p = jax.lax.broadcasted_iota(
        jnp.int32, (_C, _GG), 1).astype(jnp.float32)
    gy = jnp.floor((p + 0.5) * (1.0 / _G))
    gx = p - _G * gy

    a = pl.program_id(1)
    aw = anchor_ref[a, 0]
    ah = anchor_ref[a, 1]

    sig = jax.nn.sigmoid(xa)
    ex = jnp.exp(xa)

    is_w = cidx == 2
    is_h = cidx == 3
    base = jnp.where(is_w | is_h, ex, sig)
    add = jnp.where(cidx == 0, gx, jnp.where(cidx == 1, gy, 0.0))
    scale = jnp.where(cidx < 2, _STRIDE,
                      jnp.where(is_w, aw, jnp.where(is_h, ah, 1.0)))
    y = (base + add) * scale
    out_ref[0] = y.T


def kernel(x, anchor_wh):
    B = x.shape[0]
    out = pl.pallas_call(
        _body,
        grid=(B, 3),
        in_specs=[
            pl.BlockSpec(memory_space=pltpu.SMEM),
            pl.BlockSpec((1, _C, _G, _G), lambda b, a: (b, a, 0, 0)),
        ],
        out_specs=pl.BlockSpec((1, _GG, _C), lambda b, a: (b, a, 0)),
        out_shape=jax.ShapeDtypeStruct((B, 3 * _GG, _C), jnp.float32),
        compiler_params=pltpu.CompilerParams(
            dimension_semantics=("parallel", "arbitrary"),
        ),
    )(anchor_wh, x)
    return out


# speedup vs baseline: 1.2024x; 1.1745x over previous
import jax
import jax.numpy as jnp
from jax.experimental import pallas as pl
from jax.experimental.pallas import tpu as pltpu

_G = 52
_GG = _G * _G
_C = 25
_STRIDE = 8.0


def _body(anchor_ref, x_ref, out_ref):
    cidx = jax.lax.broadcasted_iota(jnp.int32, (_C, _GG), 0)
    p = jax.lax.broadcasted_iota(
        jnp.int32, (_C, _GG), 1).astype(jnp.float32)
    gy = jnp.floor((p + 0.5) * (1.0 / _G))
    gx = p - _G * gy
    add = jnp.where(cidx == 0, gx, jnp.where(cidx == 1, gy, 0.0))
    is_w = cidx == 2
    is_h = cidx == 3
    is_wh = is_w | is_h

    for a in range(3):
        xa = x_ref[0, a * _C:(a + 1) * _C].reshape(_C, _GG)
        aw = anchor_ref[a, 0]
        ah = anchor_ref[a, 1]
        sig = jax.nn.sigmoid(xa)
        ex = jnp.exp(xa)
        base = jnp.where(is_wh, ex, sig)
        scale = jnp.where(cidx < 2, _STRIDE,
                          jnp.where(is_w, aw, jnp.where(is_h, ah, 1.0)))
        y = (base + add) * scale
        out_ref[0, a * _GG:(a + 1) * _GG, :] = y.T


def kernel(x, anchor_wh):
    B = x.shape[0]
    out = pl.pallas_call(
        _body,
        grid=(B,),
        in_specs=[
            pl.BlockSpec(memory_space=pltpu.SMEM),
            pl.BlockSpec((1, 3 * _C, _G, _G), lambda b: (b, 0, 0, 0)),
        ],
        out_specs=pl.BlockSpec((1, 3 * _GG, _C), lambda b: (b, 0, 0)),
        out_shape=jax.ShapeDtypeStruct((B, 3 * _GG, _C), jnp.float32),
        compiler_params=pltpu.CompilerParams(
            dimension_semantics=("arbitrary",),
        ),
    )(anchor_wh, x)
    return out


# speedup vs baseline: 1.2539x; 1.0429x over previous
import jax
import jax.numpy as jnp
from jax.experimental import pallas as pl
from jax.experimental.pallas import tpu as pltpu

_G = 52
_GG = _G * _G
_C = 25
_STRIDE = 8.0
_NB = 2


def _body(anchor_ref, x_ref, out_ref):
    p = jax.lax.broadcasted_iota(
        jnp.int32, (1, _GG), 1).astype(jnp.float32)
    gy = jnp.floor((p + 0.5) * (1.0 / _G))
    gx = p - _G * gy
    grid01 = jnp.concatenate([gx, gy], axis=0)

    for n in range(_NB):
        for a in range(3):
            xa = x_ref[n, a * _C:(a + 1) * _C].reshape(_C, _GG)
            xy = (jax.nn.sigmoid(xa[0:2]) + grid01) * _STRIDE
            sc23 = jnp.concatenate(
                [jnp.full((1, 1), anchor_ref[a, 0], jnp.float32),
                 jnp.full((1, 1), anchor_ref[a, 1], jnp.float32)], axis=0)
            wh = jnp.exp(xa[2:4]) * sc23
            rest = jax.nn.sigmoid(xa[4:_C])
            y = jnp.concatenate([xy, wh, rest], axis=0)
            out_ref[n, a * _GG:(a + 1) * _GG, :] = y.T


def kernel(x, anchor_wh):
    B = x.shape[0]
    out = pl.pallas_call(
        _body,
        grid=(B // _NB,),
        in_specs=[
            pl.BlockSpec(memory_space=pltpu.SMEM),
            pl.BlockSpec((_NB, 3 * _C, _G, _G), lambda b: (b, 0, 0, 0)),
        ],
        out_specs=pl.BlockSpec((_NB, 3 * _GG, _C), lambda b: (b, 0, 0)),
        out_shape=jax.ShapeDtypeStruct((B, 3 * _GG, _C), jnp.float32),
        compiler_params=pltpu.CompilerParams(
            dimension_semantics=("arbitrary",),
        ),
    )(anchor_wh, x)
    return out


# speedup vs baseline: 1.2632x; 1.0074x over previous
import jax
import jax.numpy as jnp
from jax.experimental import pallas as pl
from jax.experimental.pallas import tpu as pltpu

_G = 52
_GG = _G * _G
_C = 25
_STRIDE = 8.0
_NB = 4


def _body(anchor_ref, x_ref, out_ref):
    p = jax.lax.broadcasted_iota(
        jnp.int32, (1, _GG), 1).astype(jnp.float32)
    gy = jnp.floor((p + 0.5) * (1.0 / _G))
    gx = p - _G * gy
    grid01 = jnp.concatenate([gx, gy], axis=0)

    for n in range(_NB):
        for a in range(3):
            xa = x_ref[n, a * _C:(a + 1) * _C].reshape(_C, _GG)
            xy = (jax.nn.sigmoid(xa[0:2]) + grid01) * _STRIDE
            sc23 = jnp.concatenate(
                [jnp.full((1, 1), anchor_ref[a, 0], jnp.float32),
                 jnp.full((1, 1), anchor_ref[a, 1], jnp.float32)], axis=0)
            wh = jnp.exp(xa[2:4]) * sc23
            rest = jax.nn.sigmoid(xa[4:_C])
            y = jnp.concatenate([xy, wh, rest], axis=0)
            out_ref[n, a * _GG:(a + 1) * _GG, :] = y.T


def kernel(x, anchor_wh):
    B = x.shape[0]
    out = pl.pallas_call(
        _body,
        grid=(B // _NB,),
        in_specs=[
            pl.BlockSpec(memory_space=pltpu.SMEM),
            pl.BlockSpec((_NB, 3 * _C, _G, _G), lambda b: (b, 0, 0, 0)),
        ],
        out_specs=pl.BlockSpec((_NB, 3 * _GG, _C), lambda b: (b, 0, 0)),
        out_shape=jax.ShapeDtypeStruct((B, 3 * _GG, _C), jnp.float32),
        compiler_params=pltpu.CompilerParams(
            dimension_semantics=("arbitrary",),
        ),
    )(anchor_wh, x)
    return out
